# Initial kernel scaffold; baseline (speedup 1.0000x reference)
import jax
import jax.numpy as jnp
from jax.experimental import pallas as pl


def kernel(features, means, W, b):
    raise NotImplementedError("write your pallas kernel here")


# speedup vs baseline: 20.1475x; 20.1475x over previous
import functools

import jax
import jax.numpy as jnp
from jax import lax
from jax.experimental import pallas as pl
from jax.experimental.pallas import tpu as pltpu
from jax.experimental.pallas import tpu_sc as plsc

_SC_CORES = 2
_SC_SUBCORES = 16
_NW = _SC_CORES * _SC_SUBCORES


def _fps_body(mx_ref, my_ref, mz_ref, out_ref, dist_ref):
    B, N = mx_ref.shape
    K = out_ref.shape[1]
    mx = mx_ref[:, :]
    my = my_ref[:, :]
    mz = mz_ref[:, :]
    col = lax.broadcasted_iota(jnp.int32, (B, N), 1)
    kcol = lax.broadcasted_iota(jnp.int32, (B, K), 1)
    boff = lax.broadcasted_iota(jnp.int32, (B, 1), 0) * N

    dist_ref[:, :] = jnp.full((B, N), jnp.inf, jnp.float32)
    neg_inf = jnp.float32(-jnp.inf)

    acc0 = jnp.broadcast_to(boff, (B, K))
    cx0 = mx[:, 0:1]
    cy0 = my[:, 0:1]
    cz0 = mz[:, 0:1]

    def body(s, carry):
        acc, cx, cy, cz = carry
        dx = mx - cx
        dy = my - cy
        dz = mz - cz
        d = dx * dx + dy * dy + dz * dz
        dist = jnp.minimum(dist_ref[:, :], d)
        dist_ref[:, :] = dist
        m = jnp.max(dist, axis=1, keepdims=True)
        far = jnp.min(jnp.where(dist == m, col, N), axis=1, keepdims=True)
        acc = jnp.where(kcol == s, jnp.broadcast_to(far + boff, (B, K)), acc)
        onehot = col == far
        cx = jnp.max(jnp.where(onehot, mx, neg_inf), axis=1, keepdims=True)
        cy = jnp.max(jnp.where(onehot, my, neg_inf), axis=1, keepdims=True)
        cz = jnp.max(jnp.where(onehot, mz, neg_inf), axis=1, keepdims=True)
        return acc, cx, cy, cz

    acc, _, _, _ = lax.fori_loop(1, K, body, (acc0, cx0, cy0, cz0))
    out_ref[:, :] = acc


def _fps_indices(mx, my, mz, K):
    B, N = mx.shape
    return pl.pallas_call(
        _fps_body,
        out_shape=jax.ShapeDtypeStruct((B, K), jnp.int32),
        scratch_shapes=[pltpu.VMEM((B, N), jnp.float32)],
    )(mx, my, mz)


def _make_sc_gather(V, D, BK):
    rows_per_w = BK // _NW
    mesh = plsc.VectorSubcoreMesh(core_axis_name="c", subcore_axis_name="s")

    @functools.partial(
        pl.kernel,
        mesh=mesh,
        out_type=jax.ShapeDtypeStruct((BK, D), jnp.float32),
        scratch_types=[
            pltpu.VMEM((rows_per_w,), jnp.int32),
            pltpu.VMEM((rows_per_w, D), jnp.float32),
            pltpu.SemaphoreType.DMA,
        ],
    )
    def gather_kernel(feat_hbm, idx_hbm, out_hbm, idx_v, rows_v, sem):
        wid = lax.axis_index("s") * _SC_CORES + lax.axis_index("c")
        base = wid * rows_per_w
        pltpu.sync_copy(idx_hbm.at[pl.ds(base, rows_per_w)], idx_v)
        pltpu.async_copy(feat_hbm.at[idx_v], rows_v, sem).wait()
        pltpu.sync_copy(rows_v, out_hbm.at[pl.ds(base, rows_per_w)])

    return gather_kernel


def _mm_body(s_ref, w_ref, b_ref, o_ref):
    o_ref[:, :] = (
        lax.dot_general(
            s_ref[:, :],
            w_ref[:, :],
            (((1,), (1,)), ((), ())),
            preferred_element_type=jnp.float32,
            precision=lax.Precision.HIGHEST,
        )
        + b_ref[:, :]
    )


def kernel(features, means, W, b):
    B, N, D = features.shape
    O = W.shape[0]
    K = min(256, N)

    mt = jnp.transpose(means, (2, 0, 1))
    gidx = _fps_indices(mt[0], mt[1], mt[2], K)

    feat_flat = features.reshape(B * N, D)
    idx_flat = gidx.reshape(B * K)
    sampled = _make_sc_gather(B * N, D, B * K)(feat_flat, idx_flat)

    out = pl.pallas_call(
        _mm_body,
        out_shape=jax.ShapeDtypeStruct((B * K, O), jnp.float32),
    )(sampled, W, b.reshape(1, O))
    return out.reshape(B, K, O)
